# baseline (speedup 1.0000x reference)
import functools

import jax
import jax.numpy as jnp
from jax import lax
from jax.experimental import pallas as pl
from jax.experimental.pallas import tpu as pltpu
from jax.experimental.pallas import tpu_sc as plsc

BATCH = 16384
EMBED = 128
D2 = EMBED // 2

NUM_CORES = 2
NUM_SUBCORES = 16
NUM_WORKERS = NUM_CORES * NUM_SUBCORES
ROWS_PER_WORKER = BATCH // NUM_WORKERS
CHUNK = 128
CHUNKS_PER_WORKER = ROWS_PER_WORKER // CHUNK
LANES = 16


def _cos_sin_body(rel_ref, out_ref):
    ph = rel_ref[...]
    out_ref[...] = jnp.concatenate([jnp.cos(ph), jnp.sin(ph)], axis=1)


def _cos_sin_table(relation_emb):
    n, d2 = relation_emb.shape
    return pl.pallas_call(
        _cos_sin_body,
        out_shape=jax.ShapeDtypeStruct((n, 2 * d2), jnp.float32),
    )(relation_emb)


def _vsqrt(s2):
    i = lax.bitcast_convert_type(s2, jnp.int32)
    i = jnp.int32(0x5F3759DF) - lax.shift_right_logical(i, 1)
    y = lax.bitcast_convert_type(i, jnp.float32)
    y = y * (1.5 - 0.5 * s2 * y * y)
    return s2 * y


def _sc_body(h_idx, r_idx, t_idx, ent, cs, out,
             idxh, idxr, idxt, hbuf0, tbuf0, csbuf0,
             hbuf1, tbuf1, csbuf1, obuf, semh, semt, semr):
    wid = lax.axis_index("s") * NUM_CORES + lax.axis_index("c")
    base = wid * ROWS_PER_WORKER
    lane15 = lax.iota(jnp.int32, LANES) == (LANES - 1)
    sets = ((hbuf0, tbuf0, csbuf0), (hbuf1, tbuf1, csbuf1))

    pltpu.sync_copy(h_idx.at[pl.ds(base, ROWS_PER_WORKER)], idxh)
    pltpu.sync_copy(t_idx.at[pl.ds(base, ROWS_PER_WORKER)], idxt)
    pltpu.sync_copy(r_idx.at[pl.ds(base, ROWS_PER_WORKER)], idxr)

    def issue(chunk):
        hbuf, tbuf, csbuf = sets[chunk % 2]
        sl = pl.ds(chunk * CHUNK, CHUNK)
        return (pltpu.async_copy(ent.at[idxh.at[sl]], hbuf, semh),
                pltpu.async_copy(ent.at[idxt.at[sl]], tbuf, semt),
                pltpu.async_copy(cs.at[idxr.at[sl]], csbuf, semr))

    pending = issue(0)
    for chunk in range(CHUNKS_PER_WORKER):
        for cp in pending:
            cp.wait()
        if chunk + 1 < CHUNKS_PER_WORKER:
            pending = issue(chunk + 1)
        hbuf, tbuf, csbuf = sets[chunk % 2]

        def row_body(r, carry, _chunk=chunk, hbuf=hbuf, tbuf=tbuf,
                     csbuf=csbuf):
            acc = jnp.zeros((LANES,), jnp.float32)
            for j in range(D2 // LANES):
                re_h = hbuf[r, pl.ds(j * LANES, LANES)]
                im_h = hbuf[r, pl.ds(D2 + j * LANES, LANES)]
                re_t = tbuf[r, pl.ds(j * LANES, LANES)]
                im_t = tbuf[r, pl.ds(D2 + j * LANES, LANES)]
                c = csbuf[r, pl.ds(j * LANES, LANES)]
                s = csbuf[r, pl.ds(D2 + j * LANES, LANES)]
                re_s = re_h * c - im_h * s - re_t
                im_s = re_h * s + im_h * c - im_t
                s2 = re_s * re_s + im_s * im_s
                acc = acc + _vsqrt(s2)
            csum = plsc.cumsum(acc)
            idx = jnp.full((LANES,), 0, jnp.int32) + (_chunk * CHUNK + r)
            plsc.store_scatter(obuf, [idx], -csum, mask=lane15)
            return carry

        lax.fori_loop(0, CHUNK, row_body, jnp.int32(0), unroll=8)

    pltpu.sync_copy(obuf, out.at[pl.ds(base, ROWS_PER_WORKER)])


@functools.partial(jax.jit, static_argnames=())
def kernel(h_idx, r_idx, t_idx, entity_emb, relation_emb):
    cs = _cos_sin_table(relation_emb)
    mesh = plsc.VectorSubcoreMesh(core_axis_name="c", subcore_axis_name="s")
    run = pl.kernel(
        _sc_body,
        out_type=jax.ShapeDtypeStruct((BATCH,), jnp.float32),
        mesh=mesh,
        compiler_params=pltpu.CompilerParams(needs_layout_passes=False),
        scratch_types=(
            [pltpu.VMEM((ROWS_PER_WORKER,), jnp.int32)] * 3
            + [pltpu.VMEM((CHUNK, EMBED), jnp.float32)] * 6
            + [pltpu.VMEM((ROWS_PER_WORKER,), jnp.float32)]
            + [pltpu.SemaphoreType.DMA] * 3
        ),
    )
    return run(h_idx.astype(jnp.int32), r_idx.astype(jnp.int32),
               t_idx.astype(jnp.int32), entity_emb, cs)


# speedup vs baseline: 1.3198x; 1.3198x over previous
import functools

import jax
import jax.numpy as jnp
from jax import lax
from jax.experimental import pallas as pl
from jax.experimental.pallas import tpu as pltpu
from jax.experimental.pallas import tpu_sc as plsc

BATCH = 16384
EMBED = 128
D2 = EMBED // 2

NUM_CORES = 2
NUM_SUBCORES = 16
NUM_WORKERS = NUM_CORES * NUM_SUBCORES
ROWS_PER_WORKER = BATCH // NUM_WORKERS
CHUNK = 128
CHUNKS_PER_WORKER = ROWS_PER_WORKER // CHUNK
LANES = 16


def _cos_sin_body(rel_ref, out_ref):
    ph = rel_ref[...]
    out_ref[...] = jnp.concatenate([jnp.cos(ph), jnp.sin(ph)], axis=1)


def _cos_sin_table(relation_emb):
    n, d2 = relation_emb.shape
    return pl.pallas_call(
        _cos_sin_body,
        out_shape=jax.ShapeDtypeStruct((n, 2 * d2), jnp.float32),
    )(relation_emb)


def _vsqrt(s2):
    i = lax.bitcast_convert_type(s2, jnp.int32)
    i = jnp.int32(0x5F3759DF) - lax.shift_right_logical(i, 1)
    y = lax.bitcast_convert_type(i, jnp.float32)
    y = y * (1.5 - 0.5 * s2 * y * y)
    return s2 * y


def _sc_body(h_idx, r_idx, t_idx, ent, cs, out,
             idxh, idxr, idxt, hbuf0, tbuf0, csbuf0,
             hbuf1, tbuf1, csbuf1, obuf, semh, semt, semr):
    wid = lax.axis_index("s") * NUM_CORES + lax.axis_index("c")
    base = wid * ROWS_PER_WORKER
    lane15 = lax.iota(jnp.int32, LANES) == (LANES - 1)
    sets = ((hbuf0, tbuf0, csbuf0), (hbuf1, tbuf1, csbuf1))

    pltpu.sync_copy(h_idx.at[pl.ds(base, ROWS_PER_WORKER)], idxh)
    pltpu.sync_copy(t_idx.at[pl.ds(base, ROWS_PER_WORKER)], idxt)
    pltpu.sync_copy(r_idx.at[pl.ds(base, ROWS_PER_WORKER)], idxr)

    def issue(chunk):
        hbuf, tbuf, csbuf = sets[chunk % 2]
        sl = pl.ds(chunk * CHUNK, CHUNK)
        return (pltpu.async_copy(ent.at[idxh.at[sl]], hbuf, semh),
                pltpu.async_copy(ent.at[idxt.at[sl]], tbuf, semt),
                pltpu.async_copy(cs.at[idxr.at[sl]], csbuf, semr))

    pending = issue(0)
    for chunk in range(CHUNKS_PER_WORKER):
        for cp in pending:
            cp.wait()
        if chunk + 1 < CHUNKS_PER_WORKER:
            pending = issue(chunk + 1)
        hbuf, tbuf, csbuf = sets[chunk % 2]

        @functools.partial(plsc.parallel_loop, 0, CHUNK, unroll=4)
        def row_body(r, _chunk=chunk, hbuf=hbuf, tbuf=tbuf,
                     csbuf=csbuf):
            acc = jnp.zeros((LANES,), jnp.float32)
            for j in range(D2 // LANES):
                re_h = hbuf[r, pl.ds(j * LANES, LANES)]
                im_h = hbuf[r, pl.ds(D2 + j * LANES, LANES)]
                re_t = tbuf[r, pl.ds(j * LANES, LANES)]
                im_t = tbuf[r, pl.ds(D2 + j * LANES, LANES)]
                c = csbuf[r, pl.ds(j * LANES, LANES)]
                s = csbuf[r, pl.ds(D2 + j * LANES, LANES)]
                re_s = re_h * c - im_h * s - re_t
                im_s = re_h * s + im_h * c - im_t
                s2 = re_s * re_s + im_s * im_s
                acc = acc + _vsqrt(s2)
            csum = plsc.cumsum(acc)
            idx = jnp.full((LANES,), 0, jnp.int32) + (_chunk * CHUNK + r)
            plsc.store_scatter(obuf, [idx], -csum, mask=lane15)

    pltpu.sync_copy(obuf, out.at[pl.ds(base, ROWS_PER_WORKER)])


@functools.partial(jax.jit, static_argnames=())
def kernel(h_idx, r_idx, t_idx, entity_emb, relation_emb):
    cs = _cos_sin_table(relation_emb)
    mesh = plsc.VectorSubcoreMesh(core_axis_name="c", subcore_axis_name="s")
    run = pl.kernel(
        _sc_body,
        out_type=jax.ShapeDtypeStruct((BATCH,), jnp.float32),
        mesh=mesh,
        compiler_params=pltpu.CompilerParams(needs_layout_passes=False),
        scratch_types=(
            [pltpu.VMEM((ROWS_PER_WORKER,), jnp.int32)] * 3
            + [pltpu.VMEM((CHUNK, EMBED), jnp.float32)] * 6
            + [pltpu.VMEM((ROWS_PER_WORKER,), jnp.float32)]
            + [pltpu.SemaphoreType.DMA] * 3
        ),
    )
    return run(h_idx.astype(jnp.int32), r_idx.astype(jnp.int32),
               t_idx.astype(jnp.int32), entity_emb, cs)
